# Initial kernel scaffold; baseline (speedup 1.0000x reference)
import jax
import jax.numpy as jnp
from jax.experimental import pallas as pl


def kernel(x):
    raise NotImplementedError("write your pallas kernel here")


# speedup vs baseline: 1.7508x; 1.7508x over previous
import functools

import jax
import jax.numpy as jnp
from jax import lax
from jax.experimental import pallas as pl
from jax.experimental.pallas import tpu as pltpu
from jax.experimental.pallas import tpu_sc as plsc

B = 4096
N = 2048
K = 256
L = 16
NW = 32
RPW = B // NW
NV = N // L
HB = 11
H1 = 1 << HB
SH1 = 31 - HB

_mesh = plsc.VectorSubcoreMesh(core_axis_name="c", subcore_axis_name="s")


@functools.partial(
    pl.kernel,
    out_type=jax.ShapeDtypeStruct((B, K * 3), jnp.float32),
    mesh=_mesh,
    compiler_params=pltpu.CompilerParams(needs_layout_passes=False),
    scratch_types=[
        pltpu.VMEM((N * 3,), jnp.float32),
        pltpu.VMEM((N,), jnp.int32),
        pltpu.VMEM((H1,), jnp.int32),
        pltpu.VMEM((N,), jnp.int32),
        pltpu.VMEM((N,), jnp.int32),
        pltpu.VMEM((N,), jnp.int32),
        pltpu.VMEM((N,), jnp.int32),
        pltpu.VMEM((256,), jnp.int32),
        pltpu.VMEM((K * 3,), jnp.float32),
    ],
)
def _topk_kernel(x_hbm, out_hbm, row_v, kd_v, hist1, ck, ci, bk, bi, hist2,
                 out_v):
    cid = lax.axis_index("c")
    sid = lax.axis_index("s")
    wid = sid * 2 + cid
    lane = lax.iota(jnp.int32, L)
    lane3 = lane * 3
    zeros = jnp.zeros((L,), jnp.int32)
    maxkd = zeros + jnp.int32(0x7FFFFFFF)

    def do_row(r, _):
        row = wid * RPW + r
        pltpu.sync_copy(x_hbm.at[row], row_v)

        def zh(i, _):
            hist1[pl.ds(i * L, L)] = zeros
            return 0

        lax.fori_loop(0, H1 // L, zh, 0)

        def nk(i, _):
            ix = lane3 + i * (3 * L)
            xx = plsc.load_gather(row_v, [ix])
            yy = plsc.load_gather(row_v, [ix + 1])
            zz = plsc.load_gather(row_v, [ix + 2])
            nrm = (xx * xx + zz * zz) + yy * yy
            kd = jnp.int32(0x7FFFFFFF) - plsc.bitcast(nrm, jnp.int32)
            kd_v[pl.ds(i * L, L)] = kd
            bb = lax.shift_right_logical(kd, SH1)
            cnt, lst = plsc.scan_count(bb)
            plsc.addupdate_scatter(hist1, [bb], cnt, mask=lst)
            return 0

        lax.fori_loop(0, NV, nk, 0)

        def cf2(i, carry):
            found, istar, base, run = carry
            h = hist1[pl.ds(i * L, L)]
            run2 = run + jnp.max(plsc.cumsum(h))
            hit = jnp.logical_and(found == 0, run2 >= K)
            istar = jnp.where(hit, i, istar)
            base = jnp.where(hit, run, base)
            found = jnp.where(hit, jnp.int32(1), found)
            return (found, istar, base, run2)

        _, istar, basecnt, _ = lax.fori_loop(
            0, H1 // L, cf2,
            (jnp.int32(0), jnp.int32(0), jnp.int32(0), jnp.int32(0)))
        h = hist1[pl.ds(istar * L, L)]
        cum = plsc.cumsum(h) + basecnt
        j = plsc.all_reduce_ffs(cum >= K)
        c = istar * L + j

        def cp(i, p):
            kd = kd_v[pl.ds(i * L, L)]
            bb = lax.shift_right_logical(kd, SH1)
            msk = bb <= c
            mi = jnp.where(msk, jnp.int32(1), jnp.int32(0))
            ics = plsc.cumsum(mi)
            pos = jnp.where(msk, p + ics - 1, jnp.int32(0))
            plsc.store_scatter(ck, [pos], kd, mask=msk)
            plsc.store_scatter(ci, [pos], lane + i * L, mask=msk)
            return p + jnp.max(ics)

        M = lax.fori_loop(0, NV, cp, jnp.int32(0))
        mp = lax.bitwise_and(M + 15, jnp.int32(~15))
        padmask = lane < (mp - M)
        padidx = jnp.minimum(M + lane, jnp.int32(N - 1))
        plsc.store_scatter(ck, [padidx], maxkd, mask=padmask)
        plsc.store_scatter(ci, [padidx], zeros, mask=padmask)
        nv = lax.shift_right_logical(mp, 4)

        for p_ in range(4):
            shift = 8 * p_
            if p_ % 2 == 0:
                src_k, src_i, dst_k, dst_i = ck, ci, bk, bi
            else:
                src_k, src_i, dst_k, dst_i = bk, bi, ck, ci
            for t in range(256 // L):
                hist2[pl.ds(t * L, L)] = zeros

            def cnt_loop(jj, _, src_k=src_k, shift=shift):
                kk = src_k[pl.ds(jj * L, L)]
                dd = lax.bitwise_and(
                    lax.shift_right_logical(kk, shift), jnp.int32(255))
                cnt, lst = plsc.scan_count(dd)
                plsc.addupdate_scatter(hist2, [dd], cnt, mask=lst)
                return 0

            lax.fori_loop(0, nv, cnt_loop, 0)

            def pf(t, run):
                hh = hist2[pl.ds(t * L, L)]
                ss = plsc.cumsum(hh)
                hist2[pl.ds(t * L, L)] = run + ss - hh
                return run + jnp.max(ss)

            lax.fori_loop(0, 256 // L, pf, jnp.int32(0))

            def sc_loop(jj, _, src_k=src_k, src_i=src_i, dst_k=dst_k,
                        dst_i=dst_i, shift=shift):
                kk = src_k[pl.ds(jj * L, L)]
                vv = src_i[pl.ds(jj * L, L)]
                dd = lax.bitwise_and(
                    lax.shift_right_logical(kk, shift), jnp.int32(255))
                cnt, lst = plsc.scan_count(dd)
                base = plsc.load_gather(hist2, [dd])
                dst = base + cnt - 1
                plsc.store_scatter(dst_k, [dst], kk)
                plsc.store_scatter(dst_i, [dst], vv)
                plsc.addupdate_scatter(hist2, [dd], cnt, mask=lst)
                return 0

            lax.fori_loop(0, nv, sc_loop, 0)

        def em(t, _):
            sidx = ci[pl.ds(t * L, L)]
            g = sidx * 3
            xx = plsc.load_gather(row_v, [g])
            yy = plsc.load_gather(row_v, [g + 1])
            zz = plsc.load_gather(row_v, [g + 2])
            o = (lane + t * L) * 3
            plsc.store_scatter(out_v, [o], xx)
            plsc.store_scatter(out_v, [o + 1], yy)
            plsc.store_scatter(out_v, [o + 2], zz)
            return 0

        lax.fori_loop(0, K // L, em, 0)
        pltpu.sync_copy(out_v, out_hbm.at[row])
        return 0

    lax.fori_loop(0, RPW, do_row, 0)


def kernel(x):
    return _topk_kernel(x).reshape(B, K, 3)


# speedup vs baseline: 4.5653x; 2.6075x over previous
import functools

import jax
import jax.numpy as jnp
from jax import lax
from jax.experimental import pallas as pl
from jax.experimental.pallas import tpu as pltpu
from jax.experimental.pallas import tpu_sc as plsc

B = 4096
N = 2048
K = 256
L = 16
NW = 32
RPW = B // NW
NV = N // L
HB = 11
H1 = 1 << HB
SH1 = 31 - HB

_mesh = plsc.VectorSubcoreMesh(core_axis_name="c", subcore_axis_name="s")


@functools.partial(
    pl.kernel,
    out_type=jax.ShapeDtypeStruct((B, K * 3), jnp.float32),
    mesh=_mesh,
    compiler_params=pltpu.CompilerParams(needs_layout_passes=False),
    scratch_types=[
        pltpu.VMEM((N * 3,), jnp.float32),
        pltpu.VMEM((N,), jnp.int32),
        pltpu.VMEM((H1,), jnp.int32),
        pltpu.VMEM((N,), jnp.int32),
        pltpu.VMEM((N,), jnp.int32),
        pltpu.VMEM((N,), jnp.int32),
        pltpu.VMEM((N,), jnp.int32),
        pltpu.VMEM((256,), jnp.int32),
        pltpu.VMEM((N,), jnp.int32),
        pltpu.VMEM((N,), jnp.int32),
        pltpu.VMEM((K * 3,), jnp.float32),
    ],
)
def _topk_kernel(x_hbm, out_hbm, row_v, kd_v, hist1, ck, ci, bk, bi, hist2,
                 dc_v, dl_v, out_v):
    cid = lax.axis_index("c")
    sid = lax.axis_index("s")
    wid = sid * 2 + cid
    lane = lax.iota(jnp.int32, L)
    lane3 = lane * 3
    lane16 = lane * L
    zeros = jnp.zeros((L,), jnp.int32)
    maxkd = zeros + jnp.int32(0x7FFFFFFF)

    @plsc.parallel_loop(0, H1 // L, unroll=4)
    def _(i):
        hist1[pl.ds(i * L, L)] = zeros

    def do_row(r, _):
        row = wid * RPW + r
        pltpu.sync_copy(x_hbm.at[row], row_v)

        @plsc.parallel_loop(0, NV, unroll=4)
        def _(i):
            ix = lane3 + i * (3 * L)
            xx = plsc.load_gather(row_v, [ix])
            yy = plsc.load_gather(row_v, [ix + 1])
            zz = plsc.load_gather(row_v, [ix + 2])
            nrm = (xx * xx + zz * zz) + yy * yy
            kd = jnp.int32(0x7FFFFFFF) - plsc.bitcast(nrm, jnp.int32)
            kd_v[pl.ds(i * L, L)] = kd
            bb = lax.shift_right_logical(kd, SH1)
            cnt, lst = plsc.scan_count(bb)
            plsc.addupdate_scatter(hist1, [bb], cnt, mask=lst)

        found = zeros
        istar = zeros
        basecnt = zeros
        run = jnp.int32(0)
        for g in range(H1 // L // L):
            acc = zeros
            for t in range(L):
                acc = acc + plsc.load_gather(hist1, [lane16 + (g * 256 + t)])
            cum = plsc.cumsum(acc) + run
            tot = jnp.max(cum)
            hit = jnp.logical_and(found == 0, tot >= K)
            jg = plsc.all_reduce_ffs(cum >= K)
            cum_j = jnp.max(cum[jg])
            acc_j = jnp.max(acc[jg])
            istar = jnp.where(hit, g * L + jg, istar)
            basecnt = jnp.where(hit, cum_j - acc_j, basecnt)
            found = jnp.where(hit, jnp.int32(1), found)
            run = tot
        istar_s = jnp.max(istar)
        h = hist1[pl.ds(istar_s * L, L)]
        cum2 = plsc.cumsum(h) + jnp.max(basecnt)
        j = plsc.all_reduce_ffs(cum2 >= K)
        c = istar_s * L + j

        @plsc.parallel_loop(0, NV, unroll=4, carry=zeros)
        def pfin(i, p):
            kd = kd_v[pl.ds(i * L, L)]
            bb = lax.shift_right_logical(kd, SH1)
            msk = bb <= c
            mi = jnp.where(msk, jnp.int32(1), jnp.int32(0))
            ics = plsc.cumsum(mi)
            pos = jnp.where(msk, p + ics - 1, jnp.int32(0))
            plsc.store_scatter(ck, [pos], kd, mask=msk)
            plsc.store_scatter(ci, [pos], lane + i * L, mask=msk)
            hist1[pl.ds(i * L, L)] = zeros
            return p + plsc.all_reduce_population_count(msk)

        M = jnp.max(pfin)
        mp = lax.bitwise_and(M + 15, jnp.int32(~15))
        padmask = lane < (mp - M)
        padidx = jnp.minimum(M + lane, jnp.int32(N - 1))
        plsc.store_scatter(ck, [padidx], maxkd, mask=padmask)
        plsc.store_scatter(ci, [padidx], zeros, mask=padmask)
        nv = lax.shift_right_logical(mp, 4)

        for p_ in range(4):
            shift = 8 * p_
            if p_ % 2 == 0:
                src_k, src_i, dst_k, dst_i = ck, ci, bk, bi
            else:
                src_k, src_i, dst_k, dst_i = bk, bi, ck, ci
            for t in range(256 // L):
                hist2[pl.ds(t * L, L)] = zeros

            @plsc.parallel_loop(0, nv, unroll=4)
            def _(jj, src_k=src_k, shift=shift):
                kk = src_k[pl.ds(jj * L, L)]
                dd = lax.bitwise_and(
                    lax.shift_right_logical(kk, shift), jnp.int32(255))
                cnt, lst = plsc.scan_count(dd)
                plsc.addupdate_scatter(hist2, [dd], cnt, mask=lst)
                dc_v[pl.ds(jj * L, L)] = cnt
                dl_v[pl.ds(jj * L, L)] = jnp.where(lst, jnp.int32(1),
                                                   jnp.int32(0))

            sums = []
            for t in range(256 // L):
                sums.append(jnp.sum(hist2[pl.ds(t * L, L)]))
            runp = jnp.int32(0)
            starts = []
            for t in range(256 // L):
                starts.append(runp)
                runp = runp + sums[t]
            for t in range(256 // L):
                hh = hist2[pl.ds(t * L, L)]
                ss = plsc.cumsum(hh)
                hist2[pl.ds(t * L, L)] = starts[t] + ss - hh

            def sc_loop(jj, _, src_k=src_k, src_i=src_i, dst_k=dst_k,
                        dst_i=dst_i, shift=shift):
                kk = src_k[pl.ds(jj * L, L)]
                vv = src_i[pl.ds(jj * L, L)]
                dd = lax.bitwise_and(
                    lax.shift_right_logical(kk, shift), jnp.int32(255))
                cnt = dc_v[pl.ds(jj * L, L)]
                lst = dl_v[pl.ds(jj * L, L)] == 1
                base = plsc.load_gather(hist2, [dd])
                dst = base + cnt - 1
                plsc.store_scatter(dst_k, [dst], kk)
                plsc.store_scatter(dst_i, [dst], vv)
                plsc.addupdate_scatter(hist2, [dd], cnt, mask=lst)
                return 0

            lax.fori_loop(0, nv, sc_loop, 0)

        @plsc.parallel_loop(0, K // L, unroll=4)
        def _(t):
            sidx = ci[pl.ds(t * L, L)]
            g = sidx * 3
            xx = plsc.load_gather(row_v, [g])
            yy = plsc.load_gather(row_v, [g + 1])
            zz = plsc.load_gather(row_v, [g + 2])
            o = lane3 + t * (3 * L)
            plsc.store_scatter(out_v, [o], xx)
            plsc.store_scatter(out_v, [o + 1], yy)
            plsc.store_scatter(out_v, [o + 2], zz)

        pltpu.sync_copy(out_v, out_hbm.at[row])
        return 0

    lax.fori_loop(0, RPW, do_row, 0)


def kernel(x):
    return _topk_kernel(x).reshape(B, K, 3)


# speedup vs baseline: 5.8878x; 1.2897x over previous
import functools

import jax
import jax.numpy as jnp
from jax import lax
from jax.experimental import pallas as pl
from jax.experimental.pallas import tpu as pltpu
from jax.experimental.pallas import tpu_sc as plsc

B = 4096
N = 2048
K = 256
L = 16
NW = 32
RPW = B // NW
NV = N // L
HB = 11
H1 = 1 << HB
SH1 = 31 - HB

_mesh = plsc.VectorSubcoreMesh(core_axis_name="c", subcore_axis_name="s")


@functools.partial(
    pl.kernel,
    out_type=jax.ShapeDtypeStruct((B, K * 3), jnp.float32),
    mesh=_mesh,
    compiler_params=pltpu.CompilerParams(needs_layout_passes=False),
    scratch_types=[
        pltpu.VMEM((2 * N * 3,), jnp.float32),
        pltpu.VMEM((N,), jnp.int32),
        pltpu.VMEM((H1,), jnp.int32),
        pltpu.VMEM((N,), jnp.int32),
        pltpu.VMEM((N,), jnp.int32),
        pltpu.VMEM((N,), jnp.int32),
        pltpu.VMEM((N,), jnp.int32),
        pltpu.VMEM((256,), jnp.int32),
        pltpu.VMEM((N,), jnp.int32),
        pltpu.VMEM((N,), jnp.int32),
        pltpu.VMEM((2 * K * 3,), jnp.float32),
        pltpu.SemaphoreType.DMA,
        pltpu.SemaphoreType.DMA,
    ],
)
def _topk_kernel(x_hbm, out_hbm, row_v, kd_v, hist1, ck, ci, bk, bi, hist2,
                 dc_v, dl_v, out_v, in_sem, out_sem):
    cid = lax.axis_index("c")
    sid = lax.axis_index("s")
    wid = sid * 2 + cid
    lane = lax.iota(jnp.int32, L)
    lane3 = lane * 3
    lane16 = lane * L
    zeros = jnp.zeros((L,), jnp.int32)
    maxkd = zeros + jnp.int32(0x7FFFFFFF)

    @plsc.parallel_loop(0, H1 // L, unroll=4)
    def _(i):
        hist1[pl.ds(i * L, L)] = zeros

    pltpu.async_copy(x_hbm.at[wid * RPW], row_v.at[pl.ds(0, N * 3)], in_sem)

    def do_row(r, _):
        row = wid * RPW + r
        cur = lax.bitwise_and(r, jnp.int32(1))
        roff = cur * (N * 3)
        pltpu.make_async_copy(
            x_hbm.at[row], row_v.at[pl.ds(roff, N * 3)], in_sem).wait()

        @pl.when(r + 1 < RPW)
        def _():
            pltpu.async_copy(
                x_hbm.at[row + 1],
                row_v.at[pl.ds((N * 3) - roff, N * 3)], in_sem)

        @plsc.parallel_loop(0, NV, unroll=4)
        def _(i):
            ix = lane3 + i * (3 * L) + roff
            xx = plsc.load_gather(row_v, [ix])
            yy = plsc.load_gather(row_v, [ix + 1])
            zz = plsc.load_gather(row_v, [ix + 2])
            nrm = (xx * xx + zz * zz) + yy * yy
            kd = jnp.int32(0x7FFFFFFF) - plsc.bitcast(nrm, jnp.int32)
            kd_v[pl.ds(i * L, L)] = kd
            bb = lax.shift_right_logical(kd, SH1)
            cnt, lst = plsc.scan_count(bb)
            plsc.addupdate_scatter(hist1, [bb], cnt, mask=lst)

        found = zeros
        istar = zeros
        basecnt = zeros
        run = jnp.int32(0)
        for g in range(H1 // L // L):
            acc = zeros
            for t in range(L):
                acc = acc + plsc.load_gather(hist1, [lane16 + (g * 256 + t)])
            cum = plsc.cumsum(acc) + run
            tot = jnp.max(cum)
            hit = jnp.logical_and(found == 0, tot >= K)
            jg = plsc.all_reduce_ffs(cum >= K)
            cum_j = jnp.max(cum[jg])
            acc_j = jnp.max(acc[jg])
            istar = jnp.where(hit, g * L + jg, istar)
            basecnt = jnp.where(hit, cum_j - acc_j, basecnt)
            found = jnp.where(hit, jnp.int32(1), found)
            run = tot
        istar_s = jnp.max(istar)
        h = hist1[pl.ds(istar_s * L, L)]
        cum2 = plsc.cumsum(h) + jnp.max(basecnt)
        j = plsc.all_reduce_ffs(cum2 >= K)
        c = istar_s * L + j

        @plsc.parallel_loop(0, NV, unroll=4, carry=zeros)
        def pfin(i, p):
            kd = kd_v[pl.ds(i * L, L)]
            bb = lax.shift_right_logical(kd, SH1)
            msk = bb <= c
            mi = jnp.where(msk, jnp.int32(1), jnp.int32(0))
            ics = plsc.cumsum(mi)
            pos = jnp.where(msk, p + ics - 1, jnp.int32(0))
            plsc.store_scatter(ck, [pos], kd, mask=msk)
            plsc.store_scatter(ci, [pos], lane + i * L, mask=msk)
            hist1[pl.ds(i * L, L)] = zeros
            return p + plsc.all_reduce_population_count(msk)

        M = jnp.max(pfin)
        mp = lax.bitwise_and(M + 15, jnp.int32(~15))
        padmask = lane < (mp - M)
        padidx = jnp.minimum(M + lane, jnp.int32(N - 1))
        plsc.store_scatter(ck, [padidx], maxkd, mask=padmask)
        plsc.store_scatter(ci, [padidx], zeros, mask=padmask)
        nv = lax.shift_right_logical(mp, 4)

        for p_ in range(4):
            shift = 8 * p_
            if p_ % 2 == 0:
                src_k, src_i, dst_k, dst_i = ck, ci, bk, bi
            else:
                src_k, src_i, dst_k, dst_i = bk, bi, ck, ci
            for t in range(256 // L):
                hist2[pl.ds(t * L, L)] = zeros

            @plsc.parallel_loop(0, nv, unroll=4)
            def _(jj, src_k=src_k, shift=shift):
                kk = src_k[pl.ds(jj * L, L)]
                dd = lax.bitwise_and(
                    lax.shift_right_logical(kk, shift), jnp.int32(255))
                cnt, lst = plsc.scan_count(dd)
                plsc.addupdate_scatter(hist2, [dd], cnt, mask=lst)
                dc_v[pl.ds(jj * L, L)] = cnt
                dl_v[pl.ds(jj * L, L)] = jnp.where(lst, jnp.int32(1),
                                                   jnp.int32(0))

            sums = []
            for t in range(256 // L):
                sums.append(jnp.sum(hist2[pl.ds(t * L, L)]))
            runp = jnp.int32(0)
            starts = []
            for t in range(256 // L):
                starts.append(runp)
                runp = runp + sums[t]
            for t in range(256 // L):
                hh = hist2[pl.ds(t * L, L)]
                ss = plsc.cumsum(hh)
                hist2[pl.ds(t * L, L)] = starts[t] + ss - hh

            def sc_loop(jj, _, src_k=src_k, src_i=src_i, dst_k=dst_k,
                        dst_i=dst_i, shift=shift):
                kk = src_k[pl.ds(jj * L, L)]
                vv = src_i[pl.ds(jj * L, L)]
                dd = lax.bitwise_and(
                    lax.shift_right_logical(kk, shift), jnp.int32(255))
                cnt = dc_v[pl.ds(jj * L, L)]
                lst = dl_v[pl.ds(jj * L, L)] == 1
                base = plsc.load_gather(hist2, [dd])
                dst = base + cnt - 1
                plsc.store_scatter(dst_k, [dst], kk)
                plsc.store_scatter(dst_i, [dst], vv)
                plsc.addupdate_scatter(hist2, [dd], cnt, mask=lst)
                return 0

            lax.fori_loop(0, nv, sc_loop, 0)

        ooff = cur * (K * 3)

        @pl.when(r >= 2)
        def _():
            pltpu.make_async_copy(
                out_v.at[pl.ds(ooff, K * 3)], out_hbm.at[row],
                out_sem).wait()

        @plsc.parallel_loop(0, K // L, unroll=4)
        def _(t):
            sidx = ci[pl.ds(t * L, L)]
            g = sidx * 3 + roff
            xx = plsc.load_gather(row_v, [g])
            yy = plsc.load_gather(row_v, [g + 1])
            zz = plsc.load_gather(row_v, [g + 2])
            o = lane3 + t * (3 * L) + ooff
            plsc.store_scatter(out_v, [o], xx)
            plsc.store_scatter(out_v, [o + 1], yy)
            plsc.store_scatter(out_v, [o + 2], zz)

        pltpu.async_copy(
            out_v.at[pl.ds(ooff, K * 3)], out_hbm.at[row], out_sem)
        return 0

    lax.fori_loop(0, RPW, do_row, 0)
    for _ in range(2):
        pltpu.make_async_copy(
            out_v.at[pl.ds(0, K * 3)], out_hbm.at[wid * RPW],
            out_sem).wait()


def kernel(x):
    return _topk_kernel(x).reshape(B, K, 3)


# speedup vs baseline: 5.9745x; 1.0147x over previous
import functools

import jax
import jax.numpy as jnp
from jax import lax
from jax.experimental import pallas as pl
from jax.experimental.pallas import tpu as pltpu
from jax.experimental.pallas import tpu_sc as plsc

B = 4096
N = 2048
K = 256
L = 16
NW = 32
RPW = B // NW
NV = N // L
HB = 11
H1 = 1 << HB
SH1 = 31 - HB

_mesh = plsc.VectorSubcoreMesh(core_axis_name="c", subcore_axis_name="s")


@functools.partial(
    pl.kernel,
    out_type=jax.ShapeDtypeStruct((B, K * 3), jnp.float32),
    mesh=_mesh,
    compiler_params=pltpu.CompilerParams(needs_layout_passes=False),
    scratch_types=[
        pltpu.VMEM((2 * N * 3,), jnp.float32),
        pltpu.VMEM((N,), jnp.int32),
        pltpu.VMEM((H1,), jnp.int32),
        pltpu.VMEM((N,), jnp.int32),
        pltpu.VMEM((N,), jnp.int32),
        pltpu.VMEM((N,), jnp.int32),
        pltpu.VMEM((N,), jnp.int32),
        pltpu.VMEM((256,), jnp.int32),
        pltpu.VMEM((N,), jnp.int32),
        pltpu.VMEM((2 * K * 3,), jnp.float32),
        pltpu.SemaphoreType.DMA,
        pltpu.SemaphoreType.DMA,
    ],
)
def _topk_kernel(x_hbm, out_hbm, row_v, kd_v, hist1, ck, ci, bk, bi, hist2,
                 dc_v, out_v, in_sem, out_sem):
    cid = lax.axis_index("c")
    sid = lax.axis_index("s")
    wid = sid * 2 + cid
    lane = lax.iota(jnp.int32, L)
    lane3 = lane * 3
    lane16 = lane * L
    zeros = jnp.zeros((L,), jnp.int32)
    maxkd = zeros + jnp.int32(0x7FFFFFFF)

    @plsc.parallel_loop(0, H1 // L, unroll=4)
    def _(i):
        hist1[pl.ds(i * L, L)] = zeros

    pltpu.async_copy(x_hbm.at[wid * RPW], row_v.at[pl.ds(0, N * 3)], in_sem)

    def do_row(r, _):
        row = wid * RPW + r
        cur = lax.bitwise_and(r, jnp.int32(1))
        roff = cur * (N * 3)
        pltpu.make_async_copy(
            x_hbm.at[row], row_v.at[pl.ds(roff, N * 3)], in_sem).wait()

        @pl.when(r + 1 < RPW)
        def _():
            pltpu.async_copy(
                x_hbm.at[row + 1],
                row_v.at[pl.ds((N * 3) - roff, N * 3)], in_sem)

        @plsc.parallel_loop(0, NV, unroll=8)
        def _(i):
            ix = lane3 + i * (3 * L) + roff
            xx = plsc.load_gather(row_v, [ix])
            yy = plsc.load_gather(row_v, [ix + 1])
            zz = plsc.load_gather(row_v, [ix + 2])
            nrm = (xx * xx + zz * zz) + yy * yy
            kd = jnp.int32(0x7FFFFFFF) - plsc.bitcast(nrm, jnp.int32)
            kd_v[pl.ds(i * L, L)] = kd
            bb = lax.shift_right_logical(kd, SH1)
            cnt, lst = plsc.scan_count(bb)
            plsc.addupdate_scatter(hist1, [bb], cnt, mask=lst)

        found = zeros
        istar = zeros
        basecnt = zeros
        run = jnp.int32(0)
        for g in range(H1 // L // L):
            acc = zeros
            for t in range(L):
                acc = acc + plsc.load_gather(hist1, [lane16 + (g * 256 + t)])
            cum = plsc.cumsum(acc) + run
            tot = jnp.max(cum)
            hit = jnp.logical_and(found == 0, tot >= K)
            jg = plsc.all_reduce_ffs(cum >= K)
            cum_j = jnp.max(cum[jg])
            acc_j = jnp.max(acc[jg])
            istar = jnp.where(hit, g * L + jg, istar)
            basecnt = jnp.where(hit, cum_j - acc_j, basecnt)
            found = jnp.where(hit, jnp.int32(1), found)
            run = tot
        istar_s = jnp.max(istar)
        h = hist1[pl.ds(istar_s * L, L)]
        cum2 = plsc.cumsum(h) + jnp.max(basecnt)
        j = plsc.all_reduce_ffs(cum2 >= K)
        c = istar_s * L + j

        @plsc.parallel_loop(0, NV, unroll=4, carry=zeros)
        def pfin(i, p):
            kd = kd_v[pl.ds(i * L, L)]
            bb = lax.shift_right_logical(kd, SH1)
            msk = bb <= c
            mi = jnp.where(msk, jnp.int32(1), jnp.int32(0))
            ics = plsc.cumsum(mi)
            pos = jnp.where(msk, p + ics - 1, jnp.int32(0))
            plsc.store_scatter(ck, [pos], kd, mask=msk)
            plsc.store_scatter(ci, [pos], lane + i * L, mask=msk)
            hist1[pl.ds(i * L, L)] = zeros
            return p + plsc.all_reduce_population_count(msk)

        M = jnp.max(pfin)
        mp = lax.bitwise_and(M + 15, jnp.int32(~15))
        padmask = lane < (mp - M)
        padidx = jnp.minimum(M + lane, jnp.int32(N - 1))
        plsc.store_scatter(ck, [padidx], maxkd, mask=padmask)
        plsc.store_scatter(ci, [padidx], zeros, mask=padmask)
        nv = lax.shift_right_logical(mp, 4)

        for p_ in range(4):
            shift = 8 * p_
            if p_ % 2 == 0:
                src_k, src_i, dst_k, dst_i = ck, ci, bk, bi
            else:
                src_k, src_i, dst_k, dst_i = bk, bi, ck, ci
            for t in range(256 // L):
                hist2[pl.ds(t * L, L)] = zeros

            @plsc.parallel_loop(0, nv, unroll=4)
            def _(jj, src_k=src_k, shift=shift):
                kk = src_k[pl.ds(jj * L, L)]
                dd = lax.bitwise_and(
                    lax.shift_right_logical(kk, shift), jnp.int32(255))
                cnt, lst = plsc.scan_count(dd)
                plsc.addupdate_scatter(hist2, [dd], cnt, mask=lst)
                dc_v[pl.ds(jj * L, L)] = cnt + jnp.where(
                    lst, jnp.int32(256), jnp.int32(0))

            sums = []
            for t in range(256 // L):
                sums.append(jnp.sum(hist2[pl.ds(t * L, L)]))
            runp = jnp.int32(0)
            starts = []
            for t in range(256 // L):
                starts.append(runp)
                runp = runp + sums[t]
            for t in range(256 // L):
                hh = hist2[pl.ds(t * L, L)]
                ss = plsc.cumsum(hh)
                hist2[pl.ds(t * L, L)] = starts[t] + ss - hh

            last_pass = p_ == 3

            def sc_loop(jj, _, src_k=src_k, src_i=src_i, dst_k=dst_k,
                        dst_i=dst_i, shift=shift, last_pass=last_pass):
                kk = src_k[pl.ds(jj * L, L)]
                vv = src_i[pl.ds(jj * L, L)]
                dd = lax.bitwise_and(
                    lax.shift_right_logical(kk, shift), jnp.int32(255))
                cl = dc_v[pl.ds(jj * L, L)]
                cnt = lax.bitwise_and(cl, jnp.int32(255))
                lst = cl >= 256
                base = plsc.load_gather(hist2, [dd])
                dst = base + cnt - 1
                if not last_pass:
                    plsc.store_scatter(dst_k, [dst], kk)
                plsc.store_scatter(dst_i, [dst], vv)
                plsc.addupdate_scatter(hist2, [dd], cnt, mask=lst)
                return 0

            lax.fori_loop(0, nv, sc_loop, 0)

        ooff = cur * (K * 3)

        @pl.when(r >= 2)
        def _():
            pltpu.make_async_copy(
                out_v.at[pl.ds(ooff, K * 3)], out_hbm.at[row],
                out_sem).wait()

        @plsc.parallel_loop(0, K // L, unroll=4)
        def _(t):
            sidx = ci[pl.ds(t * L, L)]
            g = sidx * 3 + roff
            xx = plsc.load_gather(row_v, [g])
            yy = plsc.load_gather(row_v, [g + 1])
            zz = plsc.load_gather(row_v, [g + 2])
            o = lane3 + t * (3 * L) + ooff
            plsc.store_scatter(out_v, [o], xx)
            plsc.store_scatter(out_v, [o + 1], yy)
            plsc.store_scatter(out_v, [o + 2], zz)

        pltpu.async_copy(
            out_v.at[pl.ds(ooff, K * 3)], out_hbm.at[row], out_sem)
        return 0

    lax.fori_loop(0, RPW, do_row, 0)
    for _ in range(2):
        pltpu.make_async_copy(
            out_v.at[pl.ds(0, K * 3)], out_hbm.at[wid * RPW],
            out_sem).wait()


def kernel(x):
    return _topk_kernel(x).reshape(B, K, 3)


# speedup vs baseline: 6.0633x; 1.0149x over previous
import functools

import jax
import jax.numpy as jnp
from jax import lax
from jax.experimental import pallas as pl
from jax.experimental.pallas import tpu as pltpu
from jax.experimental.pallas import tpu_sc as plsc

B = 4096
N = 2048
K = 256
L = 16
NW = 32
RPW = B // NW
NV = N // L
HB = 11
H1 = 1 << HB
SH1 = 31 - HB

_mesh = plsc.VectorSubcoreMesh(core_axis_name="c", subcore_axis_name="s")


@functools.partial(
    pl.kernel,
    out_type=jax.ShapeDtypeStruct((B, K * 3), jnp.float32),
    mesh=_mesh,
    compiler_params=pltpu.CompilerParams(needs_layout_passes=False),
    scratch_types=[
        pltpu.VMEM((2 * N * 3,), jnp.float32),
        pltpu.VMEM((N,), jnp.int32),
        pltpu.VMEM((H1,), jnp.int32),
        pltpu.VMEM((N,), jnp.int32),
        pltpu.VMEM((N,), jnp.int32),
        pltpu.VMEM((N,), jnp.int32),
        pltpu.VMEM((N,), jnp.int32),
        pltpu.VMEM((256,), jnp.int32),
        pltpu.VMEM((N,), jnp.int32),
        pltpu.VMEM((2 * K * 3,), jnp.float32),
        pltpu.SemaphoreType.DMA,
        pltpu.SemaphoreType.DMA,
    ],
)
def _topk_kernel(x_hbm, out_hbm, row_v, kd_v, hist1, ck, ci, bk, bi, hist2,
                 dc_v, out_v, in_sem, out_sem):
    cid = lax.axis_index("c")
    sid = lax.axis_index("s")
    wid = sid * 2 + cid
    lane = lax.iota(jnp.int32, L)
    lane3 = lane * 3
    lane16 = lane * L
    zeros = jnp.zeros((L,), jnp.int32)
    maxkd = zeros + jnp.int32(0x7FFFFFFF)

    @plsc.parallel_loop(0, H1 // L, unroll=4)
    def _(i):
        hist1[pl.ds(i * L, L)] = zeros

    pltpu.async_copy(x_hbm.at[wid * RPW], row_v.at[pl.ds(0, N * 3)], in_sem)

    def do_row(r, _):
        row = wid * RPW + r
        cur = lax.bitwise_and(r, jnp.int32(1))
        roff = cur * (N * 3)
        pltpu.make_async_copy(
            x_hbm.at[row], row_v.at[pl.ds(roff, N * 3)], in_sem).wait()

        @pl.when(r + 1 < RPW)
        def _():
            pltpu.async_copy(
                x_hbm.at[row + 1],
                row_v.at[pl.ds((N * 3) - roff, N * 3)], in_sem)

        @plsc.parallel_loop(0, NV, unroll=8)
        def _(i):
            ix = lane3 + i * (3 * L) + roff
            xx = plsc.load_gather(row_v, [ix])
            yy = plsc.load_gather(row_v, [ix + 1])
            zz = plsc.load_gather(row_v, [ix + 2])
            nrm = (xx * xx + zz * zz) + yy * yy
            kd = jnp.int32(0x7FFFFFFF) - plsc.bitcast(nrm, jnp.int32)
            kd_v[pl.ds(i * L, L)] = kd
            bb = lax.shift_right_logical(kd, SH1)
            cnt, lst = plsc.scan_count(bb)
            plsc.addupdate_scatter(hist1, [bb], cnt, mask=lst)

        found = zeros
        istar = zeros
        basecnt = zeros
        run = jnp.int32(0)
        for g in range(H1 // L // L):
            acc_a = zeros
            acc_b = zeros
            for t in range(L // 2):
                acc_a = acc_a + plsc.load_gather(hist1,
                                                 [lane16 + (g * 256 + t)])
                acc_b = acc_b + plsc.load_gather(
                    hist1, [lane16 + (g * 256 + t + L // 2)])
            acc = acc_a + acc_b
            cum = plsc.cumsum(acc) + run
            tot = jnp.max(cum)
            hit = jnp.logical_and(found == 0, tot >= K)
            jg = plsc.all_reduce_ffs(cum >= K)
            cum_j = jnp.max(cum[jg])
            acc_j = jnp.max(acc[jg])
            istar = jnp.where(hit, g * L + jg, istar)
            basecnt = jnp.where(hit, cum_j - acc_j, basecnt)
            found = jnp.where(hit, jnp.int32(1), found)
            run = tot
        istar_s = jnp.max(istar)
        h = hist1[pl.ds(istar_s * L, L)]
        cum2 = plsc.cumsum(h) + jnp.max(basecnt)
        j = plsc.all_reduce_ffs(cum2 >= K)
        c = istar_s * L + j

        @plsc.parallel_loop(0, NV, unroll=8, carry=zeros)
        def pfin(i, p):
            kd = kd_v[pl.ds(i * L, L)]
            bb = lax.shift_right_logical(kd, SH1)
            msk = bb <= c
            mi = jnp.where(msk, jnp.int32(1), jnp.int32(0))
            ics = plsc.cumsum(mi)
            pos = jnp.where(msk, p + ics - 1, jnp.int32(0))
            plsc.store_scatter(ck, [pos], kd, mask=msk)
            plsc.store_scatter(ci, [pos], lane + i * L, mask=msk)
            hist1[pl.ds(i * L, L)] = zeros
            return p + plsc.all_reduce_population_count(msk)

        M = jnp.max(pfin)
        mp = lax.bitwise_and(M + 15, jnp.int32(~15))
        padmask = lane < (mp - M)
        padidx = jnp.minimum(M + lane, jnp.int32(N - 1))
        plsc.store_scatter(ck, [padidx], maxkd, mask=padmask)
        plsc.store_scatter(ci, [padidx], zeros, mask=padmask)
        nv = lax.shift_right_logical(mp, 4)

        for p_ in range(4):
            shift = 8 * p_
            if p_ % 2 == 0:
                src_k, src_i, dst_k, dst_i = ck, ci, bk, bi
            else:
                src_k, src_i, dst_k, dst_i = bk, bi, ck, ci
            for t in range(256 // L):
                hist2[pl.ds(t * L, L)] = zeros

            @plsc.parallel_loop(0, nv, unroll=8)
            def _(jj, src_k=src_k, shift=shift):
                kk = src_k[pl.ds(jj * L, L)]
                dd = lax.bitwise_and(
                    lax.shift_right_logical(kk, shift), jnp.int32(255))
                cnt, lst = plsc.scan_count(dd)
                plsc.addupdate_scatter(hist2, [dd], cnt, mask=lst)
                dc_v[pl.ds(jj * L, L)] = cnt + jnp.where(
                    lst, jnp.int32(256), jnp.int32(0))

            sums = []
            for t in range(256 // L):
                sums.append(jnp.sum(hist2[pl.ds(t * L, L)]))
            runp = jnp.int32(0)
            starts = []
            for t in range(256 // L):
                starts.append(runp)
                runp = runp + sums[t]
            for t in range(256 // L):
                hh = hist2[pl.ds(t * L, L)]
                ss = plsc.cumsum(hh)
                hist2[pl.ds(t * L, L)] = starts[t] + ss - hh

            last_pass = p_ == 3

            def sc_loop(jj, _, src_k=src_k, src_i=src_i, dst_k=dst_k,
                        dst_i=dst_i, shift=shift, last_pass=last_pass):
                kk = src_k[pl.ds(jj * L, L)]
                vv = src_i[pl.ds(jj * L, L)]
                dd = lax.bitwise_and(
                    lax.shift_right_logical(kk, shift), jnp.int32(255))
                cl = dc_v[pl.ds(jj * L, L)]
                cnt = lax.bitwise_and(cl, jnp.int32(255))
                lst = cl >= 256
                base = plsc.load_gather(hist2, [dd])
                dst = base + cnt - 1
                if not last_pass:
                    plsc.store_scatter(dst_k, [dst], kk)
                plsc.store_scatter(dst_i, [dst], vv)
                plsc.addupdate_scatter(hist2, [dd], cnt, mask=lst)
                return 0

            lax.fori_loop(0, nv, sc_loop, 0)

        ooff = cur * (K * 3)

        @pl.when(r >= 2)
        def _():
            pltpu.make_async_copy(
                out_v.at[pl.ds(ooff, K * 3)], out_hbm.at[row],
                out_sem).wait()

        @plsc.parallel_loop(0, K // L, unroll=8)
        def _(t):
            sidx = ci[pl.ds(t * L, L)]
            g = sidx * 3 + roff
            xx = plsc.load_gather(row_v, [g])
            yy = plsc.load_gather(row_v, [g + 1])
            zz = plsc.load_gather(row_v, [g + 2])
            o = lane3 + t * (3 * L) + ooff
            plsc.store_scatter(out_v, [o], xx)
            plsc.store_scatter(out_v, [o + 1], yy)
            plsc.store_scatter(out_v, [o + 2], zz)

        pltpu.async_copy(
            out_v.at[pl.ds(ooff, K * 3)], out_hbm.at[row], out_sem)
        return 0

    lax.fori_loop(0, RPW, do_row, 0)
    for _ in range(2):
        pltpu.make_async_copy(
            out_v.at[pl.ds(0, K * 3)], out_hbm.at[wid * RPW],
            out_sem).wait()


def kernel(x):
    return _topk_kernel(x).reshape(B, K, 3)
